# baseline (speedup 1.0000x reference)
import functools

import jax
import jax.numpy as jnp
from jax import lax
from jax.experimental import pallas as pl
from jax.experimental.pallas import tpu as pltpu
from jax.experimental.pallas import tpu_sc as plsc

BATCH = 16384
EMBED_DIM = 64
NUM_WORKERS = 32
BPW = BATCH // NUM_WORKERS
NUM_CORES = 2
CHUNK = 256
NCHUNK = BPW // CHUNK


def _bprmf_body(uq_hbm, iq_hbm, jq_hbm, uh_hbm, ih_hbm, jh_hbm,
                uemb_hbm, iemb_hbm,
                pos_out, neg_out,
                uq_v, iq_v, jq_v, uh_v, ih_v, jh_v,
                u_rows, i_rows, j_rows,
                pos_v, neg_v, sem):
    wid = lax.axis_index("s") * NUM_CORES + lax.axis_index("c")
    base = wid * BPW

    pltpu.sync_copy(uq_hbm.at[pl.ds(base, BPW)], uq_v)
    pltpu.sync_copy(iq_hbm.at[pl.ds(base, BPW)], iq_v)
    pltpu.sync_copy(jq_hbm.at[pl.ds(base, BPW)], jq_v)
    pltpu.sync_copy(uh_hbm.at[pl.ds(base, BPW)], uh_v)
    pltpu.sync_copy(ih_hbm.at[pl.ds(base, BPW)], ih_v)
    pltpu.sync_copy(jh_hbm.at[pl.ds(base, BPW)], jh_v)

    lanes = lax.iota(jnp.int32, 16)

    for c in range(NCHUNK):
        co = c * CHUNK
        cu = pltpu.async_copy(uemb_hbm.at[uq_v.at[pl.ds(co, CHUNK)]],
                              u_rows, sem)
        ci = pltpu.async_copy(iemb_hbm.at[iq_v.at[pl.ds(co, CHUNK)]],
                              i_rows, sem)
        cj = pltpu.async_copy(iemb_hbm.at[jq_v.at[pl.ds(co, CHUNK)]],
                              j_rows, sem)
        cu.wait()
        ci.wait()
        cj.wait()

        def group(g, carry, co=co):
            b0 = g * 16
            p_acc = jnp.zeros((16,), jnp.float32)
            n_acc = jnp.zeros((16,), jnp.float32)
            hu_v = uh_v[pl.ds(co + b0, 16)] * 64
            hi_v = ih_v[pl.ds(co + b0, 16)] * 64
            hj_v = jh_v[pl.ds(co + b0, 16)] * 64
            for b in range(16):
                ou = hu_v[b]
                oi = hi_v[b]
                oj = hj_v[b]
                u0 = u_rows[b0 + b, pl.ds(ou, 16)]
                u1 = u_rows[b0 + b, pl.ds(ou + 16, 16)]
                u2 = u_rows[b0 + b, pl.ds(ou + 32, 16)]
                u3 = u_rows[b0 + b, pl.ds(ou + 48, 16)]
                i0 = i_rows[b0 + b, pl.ds(oi, 16)]
                i1 = i_rows[b0 + b, pl.ds(oi + 16, 16)]
                i2 = i_rows[b0 + b, pl.ds(oi + 32, 16)]
                i3 = i_rows[b0 + b, pl.ds(oi + 48, 16)]
                j0 = j_rows[b0 + b, pl.ds(oj, 16)]
                j1 = j_rows[b0 + b, pl.ds(oj + 16, 16)]
                j2 = j_rows[b0 + b, pl.ds(oj + 32, 16)]
                j3 = j_rows[b0 + b, pl.ds(oj + 48, 16)]
                p = (u0 * i0 + u1 * i1) + (u2 * i2 + u3 * i3)
                n = (u0 * j0 + u1 * j1) + (u2 * j2 + u3 * j3)
                sel = lanes == b
                p_acc = jnp.where(sel, jnp.sum(p), p_acc)
                n_acc = jnp.where(sel, jnp.sum(n), n_acc)
            pos_v[pl.ds(co + b0, 16)] = p_acc
            neg_v[pl.ds(co + b0, 16)] = n_acc
            return carry

        lax.fori_loop(0, CHUNK // 16, group, 0)

    pltpu.sync_copy(pos_v, pos_out.at[pl.ds(base, BPW)])
    pltpu.sync_copy(neg_v, neg_out.at[pl.ds(base, BPW)])


@jax.jit
def kernel(user, pos_item, neg_item, user_emb, item_emb):
    uemb2 = user_emb.reshape(user_emb.shape[0] // 2, 2 * user_emb.shape[1])
    iemb2 = item_emb.reshape(item_emb.shape[0] // 2, 2 * item_emb.shape[1])
    uq = jnp.right_shift(user, 1)
    iq = jnp.right_shift(pos_item, 1)
    jq = jnp.right_shift(neg_item, 1)
    uh = jnp.bitwise_and(user, 1)
    ih = jnp.bitwise_and(pos_item, 1)
    jh = jnp.bitwise_and(neg_item, 1)
    mesh = plsc.VectorSubcoreMesh(core_axis_name="c", subcore_axis_name="s")
    f = pl.kernel(
        _bprmf_body,
        mesh=mesh,
        compiler_params=pltpu.CompilerParams(needs_layout_passes=False),
        out_type=(
            jax.ShapeDtypeStruct((BATCH,), jnp.float32),
            jax.ShapeDtypeStruct((BATCH,), jnp.float32),
        ),
        scratch_types=[
            pltpu.VMEM((BPW,), jnp.int32),
            pltpu.VMEM((BPW,), jnp.int32),
            pltpu.VMEM((BPW,), jnp.int32),
            pltpu.VMEM((BPW,), jnp.int32),
            pltpu.VMEM((BPW,), jnp.int32),
            pltpu.VMEM((BPW,), jnp.int32),
            pltpu.VMEM((CHUNK, 2 * EMBED_DIM), jnp.float32),
            pltpu.VMEM((CHUNK, 2 * EMBED_DIM), jnp.float32),
            pltpu.VMEM((CHUNK, 2 * EMBED_DIM), jnp.float32),
            pltpu.VMEM((BPW,), jnp.float32),
            pltpu.VMEM((BPW,), jnp.float32),
            pltpu.SemaphoreType.DMA,
        ],
    )
    return f(uq, iq, jq, uh, ih, jh, uemb2, iemb2)


# speedup vs baseline: 1.5846x; 1.5846x over previous
import functools

import jax
import jax.numpy as jnp
from jax import lax
from jax.experimental import pallas as pl
from jax.experimental.pallas import tpu as pltpu
from jax.experimental.pallas import tpu_sc as plsc

BATCH = 16384
EMBED_DIM = 64
NUM_WORKERS = 32
BPW = BATCH // NUM_WORKERS
NUM_CORES = 2
CHUNK = 256
NCHUNK = BPW // CHUNK


def _bprmf_body(user_hbm, pos_hbm, neg_hbm, uemb_hbm, iemb_hbm,
                pos_out, neg_out,
                uq_v, iq_v, jq_v,
                u_rows, i_rows, j_rows,
                pos_v, neg_v, sem):
    wid = lax.axis_index("s") * NUM_CORES + lax.axis_index("c")
    base = wid * BPW

    pltpu.sync_copy(user_hbm.at[pl.ds(base, BPW)], uq_v)
    pltpu.sync_copy(pos_hbm.at[pl.ds(base, BPW)], iq_v)
    pltpu.sync_copy(neg_hbm.at[pl.ds(base, BPW)], jq_v)

    lanes = lax.iota(jnp.int32, 16)

    for c in range(NCHUNK):
        co = c * CHUNK

        def fire(g, carry, co=co):
            b0 = g * 16
            ru = uq_v[pl.ds(co + b0, 16)]
            ri = iq_v[pl.ds(co + b0, 16)]
            rj = jq_v[pl.ds(co + b0, 16)]
            for b in range(16):
                pltpu.async_copy(uemb_hbm.at[pl.ds(ru[b], 1)],
                                 u_rows.at[pl.ds(b0 + b, 1)], sem)
                pltpu.async_copy(iemb_hbm.at[pl.ds(ri[b], 1)],
                                 i_rows.at[pl.ds(b0 + b, 1)], sem)
                pltpu.async_copy(iemb_hbm.at[pl.ds(rj[b], 1)],
                                 j_rows.at[pl.ds(b0 + b, 1)], sem)
            return carry

        lax.fori_loop(0, CHUNK // 16, fire, 0)

        pltpu.make_async_copy(uemb_hbm.at[pl.ds(0, CHUNK)], u_rows, sem).wait()
        pltpu.make_async_copy(uemb_hbm.at[pl.ds(0, CHUNK)], i_rows, sem).wait()
        pltpu.make_async_copy(uemb_hbm.at[pl.ds(0, CHUNK)], j_rows, sem).wait()

        def group(g, carry, co=co):
            b0 = g * 16
            p_acc = jnp.zeros((16,), jnp.float32)
            n_acc = jnp.zeros((16,), jnp.float32)
            for b in range(16):
                u0 = u_rows[b0 + b, pl.ds(0, 16)]
                u1 = u_rows[b0 + b, pl.ds(16, 16)]
                u2 = u_rows[b0 + b, pl.ds(32, 16)]
                u3 = u_rows[b0 + b, pl.ds(48, 16)]
                i0 = i_rows[b0 + b, pl.ds(0, 16)]
                i1 = i_rows[b0 + b, pl.ds(16, 16)]
                i2 = i_rows[b0 + b, pl.ds(32, 16)]
                i3 = i_rows[b0 + b, pl.ds(48, 16)]
                j0 = j_rows[b0 + b, pl.ds(0, 16)]
                j1 = j_rows[b0 + b, pl.ds(16, 16)]
                j2 = j_rows[b0 + b, pl.ds(32, 16)]
                j3 = j_rows[b0 + b, pl.ds(48, 16)]
                p = (u0 * i0 + u1 * i1) + (u2 * i2 + u3 * i3)
                n = (u0 * j0 + u1 * j1) + (u2 * j2 + u3 * j3)
                sel = lanes == b
                p_acc = jnp.where(sel, jnp.sum(p), p_acc)
                n_acc = jnp.where(sel, jnp.sum(n), n_acc)
            pos_v[pl.ds(co + b0, 16)] = p_acc
            neg_v[pl.ds(co + b0, 16)] = n_acc
            return carry

        lax.fori_loop(0, CHUNK // 16, group, 0)

    pltpu.sync_copy(pos_v, pos_out.at[pl.ds(base, BPW)])
    pltpu.sync_copy(neg_v, neg_out.at[pl.ds(base, BPW)])


@jax.jit
def kernel(user, pos_item, neg_item, user_emb, item_emb):
    mesh = plsc.VectorSubcoreMesh(core_axis_name="c", subcore_axis_name="s")
    f = pl.kernel(
        _bprmf_body,
        mesh=mesh,
        compiler_params=pltpu.CompilerParams(needs_layout_passes=False),
        out_type=(
            jax.ShapeDtypeStruct((BATCH,), jnp.float32),
            jax.ShapeDtypeStruct((BATCH,), jnp.float32),
        ),
        scratch_types=[
            pltpu.VMEM((BPW,), jnp.int32),
            pltpu.VMEM((BPW,), jnp.int32),
            pltpu.VMEM((BPW,), jnp.int32),
            pltpu.VMEM((CHUNK, EMBED_DIM), jnp.float32),
            pltpu.VMEM((CHUNK, EMBED_DIM), jnp.float32),
            pltpu.VMEM((CHUNK, EMBED_DIM), jnp.float32),
            pltpu.VMEM((BPW,), jnp.float32),
            pltpu.VMEM((BPW,), jnp.float32),
            pltpu.SemaphoreType.DMA,
        ],
    )
    return f(user, pos_item, neg_item, user_emb, item_emb)
